# Initial kernel scaffold; baseline (speedup 1.0000x reference)
import jax
import jax.numpy as jnp
from jax.experimental import pallas as pl


def kernel(h, edge_index, W, b, att_W, att_b):
    raise NotImplementedError("write your pallas kernel here")


# speedup vs baseline: 6.1894x; 6.1894x over previous
import functools

import jax
import jax.numpy as jnp
from jax import lax
from jax.experimental import pallas as pl
from jax.experimental.pallas import tpu as pltpu
from jax.experimental.pallas import tpu_sc as plsc

N = 10000
E = 320000
D = 128
NC = 2
NS = 16
NW = NC * NS
EPT = E // NW
BC = 80
NCHUNK = EPT // BC
N_PAD = 10240
RPT = N_PAD // NS
ZROWS = 128

_f32 = jnp.float32


def _scores_body(h_ref, w_ref, a8_ref, b_ref, attb_ref, out_ref):
    vt = lax.dot_general(a8_ref[...], w_ref[...], (((1,), (0,)), ((), ())),
                         preferred_element_type=_f32)
    s = lax.dot_general(vt, h_ref[...], (((1,), (1,)), ((), ())),
                        preferred_element_type=_f32)
    cvec = lax.dot_general(a8_ref[...], b_ref[...], (((1,), (0,)), ((), ())),
                           preferred_element_type=_f32)
    row = lax.broadcasted_iota(jnp.int32, (8, 1), 0)
    cvec = cvec + jnp.where(row == 0, attb_ref[...], 0.0)
    out_ref[...] = s + cvec


def _scores(h, W, a8, b2, attb):
    return pl.pallas_call(
        _scores_body,
        out_shape=jax.ShapeDtypeStruct((8, N), _f32),
    )(h, W, a8, b2, attb)


def _edge_body(s2, srch, dsth, e_out, ms_out, ss_out,
               tabs, tabd, srcv, dstv, ev, statv):
    c = lax.axis_index("c")
    s = lax.axis_index("s")
    wid = s * NC + c
    base = wid * EPT
    pltpu.sync_copy(s2.at[0], tabs)
    pltpu.sync_copy(s2.at[1], tabd)
    pltpu.sync_copy(srch.at[pl.ds(base, EPT)], srcv)
    pltpu.sync_copy(dsth.at[pl.ds(base, EPT)], dstv)

    def score16(i, m):
        a = plsc.load_gather(tabs, [srcv[pl.ds(i * 16, 16)]])
        bb = plsc.load_gather(tabd, [dstv[pl.ds(i * 16, 16)]])
        z = a + bb
        e16 = jnp.maximum(z, 0.01 * z)
        ev[pl.ds(i * 16, 16)] = e16
        return jnp.maximum(m, e16)

    m = lax.fori_loop(0, EPT // 16, score16,
                      jnp.full((16,), -jnp.inf, _f32))
    mt = jnp.max(m)
    mv = jnp.full((16,), mt, _f32)

    def sum16(i, acc):
        return acc + jnp.exp(ev[pl.ds(i * 16, 16)] - mv)

    sv = lax.fori_loop(0, EPT // 16, sum16, jnp.zeros((16,), _f32))
    st = jnp.sum(sv)

    pltpu.sync_copy(ev, e_out.at[pl.ds(base, EPT)])
    statv[...] = mv
    pltpu.sync_copy(statv, ms_out.at[wid])
    statv[...] = jnp.full((16,), st, _f32)
    pltpu.sync_copy(statv, ss_out.at[wid])


def _edge_scores(s2, src, dst):
    mesh = plsc.VectorSubcoreMesh(core_axis_name="c", subcore_axis_name="s")
    fn = pl.kernel(
        _edge_body,
        out_type=[
            jax.ShapeDtypeStruct((E,), _f32),
            jax.ShapeDtypeStruct((NW, 16), _f32),
            jax.ShapeDtypeStruct((NW, 16), _f32),
        ],
        mesh=mesh,
        compiler_params=pltpu.CompilerParams(needs_layout_passes=False),
        scratch_types=[
            pltpu.VMEM((N,), _f32),
            pltpu.VMEM((N,), _f32),
            pltpu.VMEM((EPT,), jnp.int32),
            pltpu.VMEM((EPT,), jnp.int32),
            pltpu.VMEM((EPT,), _f32),
            pltpu.VMEM((16,), _f32),
        ],
    )
    return fn(s2, src, dst)


def _scatter_body(h, srch, dsth, eh, ms, ss, part,
                  acc, msv, ssv, srcv, dstv, ev, uv, rows, zbuf, sem):
    c = lax.axis_index("c")
    s = lax.axis_index("s")
    wid = s * NC + c
    base = wid * EPT

    pltpu.sync_copy(ms, msv)
    pltpu.sync_copy(ss, ssv)

    def mred(i, m):
        return jnp.maximum(m, msv[i, :])

    M = lax.fori_loop(0, NW, mred, jnp.full((16,), -jnp.inf, _f32))

    def sred(i, a):
        return a + ssv[i, :] * jnp.exp(msv[i, :] - M)

    S = lax.fori_loop(0, NW, sred, jnp.zeros((16,), _f32))
    invS = 1.0 / S

    def zrow(r, _):
        for j in range(8):
            zbuf[r, pl.ds(j * 16, 16)] = jnp.zeros((16,), _f32)
        return 0

    lax.fori_loop(0, ZROWS, zrow, 0)
    for k in range(RPT // ZROWS):
        pltpu.sync_copy(zbuf, acc.at[pl.ds(s * RPT + k * ZROWS, ZROWS)])
    plsc.subcore_barrier()

    def chunk(ci, _):
        off = base + ci * BC
        pltpu.sync_copy(srch.at[pl.ds(off, BC)], srcv)
        pltpu.sync_copy(dsth.at[pl.ds(off, BC)], dstv)
        pltpu.sync_copy(eh.at[pl.ds(off, BC)], ev)
        pltpu.async_copy(h.at[srcv], rows, sem).wait()
        for g in range(BC // 16):
            uv[pl.ds(g * 16, 16)] = (
                jnp.exp(ev[pl.ds(g * 16, 16)] - M) * invS)

        def rowscale(bi, _2):
            ub = plsc.load_gather(uv, [jnp.full((16,), bi, jnp.int32)])
            for j in range(8):
                rows[bi, pl.ds(j * 16, 16)] = rows[bi, pl.ds(j * 16, 16)] * ub
            return 0

        lax.fori_loop(0, BC, rowscale, 0)
        pltpu.sync_copy(rows, acc.at[dstv], add=True)
        return 0

    lax.fori_loop(0, NCHUNK, chunk, 0)
    plsc.subcore_barrier()

    for k in range(RPT // ZROWS):
        r0 = s * RPT + k * ZROWS
        pltpu.sync_copy(acc.at[pl.ds(r0, ZROWS)], part.at[c, pl.ds(r0, ZROWS)])


def _scatter(h, src, dst, e, ms, ss):
    mesh = plsc.VectorSubcoreMesh(core_axis_name="c", subcore_axis_name="s")
    fn = pl.kernel(
        _scatter_body,
        out_type=jax.ShapeDtypeStruct((NC, N_PAD, D), _f32),
        mesh=mesh,
        compiler_params=pltpu.CompilerParams(needs_layout_passes=False),
        scratch_types=[
            pltpu.VMEM_SHARED((N_PAD, D), _f32),
            pltpu.VMEM((NW, 16), _f32),
            pltpu.VMEM((NW, 16), _f32),
            pltpu.VMEM((BC,), jnp.int32),
            pltpu.VMEM((BC,), jnp.int32),
            pltpu.VMEM((BC,), _f32),
            pltpu.VMEM((BC,), _f32),
            pltpu.VMEM((BC, D), _f32),
            pltpu.VMEM((ZROWS, D), _f32),
            pltpu.SemaphoreType.DMA,
        ],
    )
    return fn(h, src, dst, e, ms, ss)


def _combine_body(p_ref, o_ref):
    o_ref[...] = jnp.maximum(p_ref[0] + p_ref[1], 0.0)


def _combine(part):
    nb = 10
    rb = N // nb
    return pl.pallas_call(
        _combine_body,
        grid=(nb,),
        in_specs=[pl.BlockSpec((NC, rb, D), lambda i: (0, i, 0))],
        out_specs=pl.BlockSpec((rb, D), lambda i: (i, 0)),
        out_shape=jax.ShapeDtypeStruct((N, D), _f32),
    )(part)


def kernel(h, edge_index, W, b, att_W, att_b):
    src = edge_index[0]
    dst = edge_index[1]
    a2rows = att_W.reshape(2, D)
    a8 = jnp.zeros((8, D), _f32).at[:2].set(a2rows)
    b2 = b.reshape(D, 1)
    attb = att_b.reshape(1, 1)

    s2 = _scores(h, W, a8, b2, attb)
    e, ms, ss = _edge_scores(s2, src, dst)
    part = _scatter(h, src, dst, e, ms, ss)
    return _combine(part)
